# baseline (speedup 1.0000x reference)
import functools

import jax
import jax.numpy as jnp
from jax import lax
from jax.experimental import pallas as pl
from jax.experimental.pallas import tpu as pltpu
from jax.experimental.pallas import tpu_sc as plsc

D = 64
BATCH = 4096
HIST = 200
HB = BATCH // 2
B = BATCH * HIST
SPLITS = (16, 84, 84, 16)
assert sum(SPLITS) == HIST and all(s % 2 == 0 for s in SPLITS)
NSPLIT = len(SPLITS)
H0S = tuple(sum(SPLITS[:i]) for i in range(NSPLIT))
NC, NS = 2, 16
NW = NC * NS
CH = 256
assert HB % CH == 0


def _ring_depth(nch):
    for nb in (6, 5, 4, 3, 2):
        if nch % nb == 0:
            return nb
    return 1

_mesh = plsc.VectorSubcoreMesh(core_axis_name="c", subcore_axis_name="s")


def _make_sc_gather(h0, hn):
    bs = hn * BATCH
    b_per_w = bs // NW
    nch = b_per_w // CH
    assert b_per_w % CH == 0
    nb = _ring_depth(nch)

    @functools.partial(
        pl.kernel,
        mesh=_mesh,
        out_type=jax.ShapeDtypeStruct((bs // 2, 2 * D), jnp.float32),
        scratch_types=[
            pltpu.VMEM((b_per_w,), jnp.int32),
        ]
        + [pltpu.VMEM((CH, D), jnp.float32)] * nb
        + [pltpu.SemaphoreType.DMA] * (2 * nb),
        compiler_params=pltpu.CompilerParams(use_tc_tiling_on_sc=False),
    )
    def _sc_gather(idx_hbm, table_hbm, out_hbm, idx_all, *bufs):
        rows = bufs[:nb]
        sg = bufs[nb:2 * nb]
        sw = bufs[2 * nb:3 * nb]

        wid = lax.axis_index("s") * NC + lax.axis_index("c")
        base = h0 * BATCH + wid * b_per_w

        pltpu.sync_copy(idx_hbm.at[pl.ds(base, b_per_w)], idx_all)

        def gather_desc(c, b):
            return pltpu.make_async_copy(
                table_hbm.at[idx_all.at[pl.ds(c * CH, CH)]], rows[b], sg[b])

        def write_desc(c, b):
            q0 = base + c * CH
            h = q0 // BATCH - h0
            p0 = q0 % BATCH
            s = p0 // HB
            r0_ = p0 % HB
            drow = pl.multiple_of(h * HB + r0_, CH)
            dcol = pl.multiple_of(s * D, D)
            return pltpu.make_async_copy(
                rows[b], out_hbm.at[pl.ds(drow, CH), pl.ds(dcol, D)], sw[b])

        for b in range(nb):
            gather_desc(b, b).start()

        def body(it, carry):
            g = it * nb
            for b in range(nb):
                c = g + b
                gather_desc(c, b).wait()
                write_desc(c, b).start()
            for b in range(nb):
                c = g + b
                write_desc(c, b).wait()

                @pl.when(c + nb < nch)
                def _():
                    gather_desc(c + nb, b).start()

            return carry

        lax.fori_loop(0, nch // nb, body, 0)

    return _sc_gather


_sc_gather_halves = tuple(_make_sc_gather(H0S[i], SPLITS[i]) for i in range(NSPLIT))


def _tc_transpose_body_first(g_ref, o_ref):
    xt = g_ref[...].T
    o_ref[0, :, 0:HB] = xt[0:D, 0:HB]
    o_ref[0, :, HB:BATCH] = xt[D:2 * D, 0:HB]
    o_ref[1, :, 0:HB] = xt[0:D, HB:BATCH]
    o_ref[1, :, HB:BATCH] = xt[D:2 * D, HB:BATCH]


def _tc_transpose_body_rest(g_ref, _prev_ref, o_ref):
    _tc_transpose_body_first(g_ref, o_ref)


def _make_tc_transpose(split, first):
    kb = H0S[split] // 2
    grid = SPLITS[split] // 2
    if first:
        return pl.pallas_call(
            _tc_transpose_body_first,
            grid=(grid,),
            in_specs=[pl.BlockSpec((BATCH, 2 * D), lambda k: (k, 0))],
            out_specs=pl.BlockSpec((2, D, BATCH), lambda k: (k + kb, 0, 0)),
            out_shape=jax.ShapeDtypeStruct((HIST, D, BATCH), jnp.float32),
        )
    return pl.pallas_call(
        _tc_transpose_body_rest,
        grid=(grid,),
        in_specs=[
            pl.BlockSpec((BATCH, 2 * D), lambda k: (k, 0)),
            pl.BlockSpec(memory_space=pl.ANY),
        ],
        out_specs=pl.BlockSpec((2, D, BATCH), lambda k: (k + kb, 0, 0)),
        out_shape=jax.ShapeDtypeStruct((HIST, D, BATCH), jnp.float32),
        input_output_aliases={1: 0},
    )


_tc_first = _make_tc_transpose(0, True)
_tc_rest = tuple(_make_tc_transpose(i, False) for i in range(1, NSPLIT))


def kernel(matched_word_ids, word_embedding_weight):
    idx_t = matched_word_ids.T.reshape(-1).astype(jnp.int32)
    g2s = [f(idx_t, word_embedding_weight) for f in _sc_gather_halves]
    out_t = _tc_first(g2s[0])
    for i in range(1, NSPLIT):
        out_t = _tc_rest[i - 1](g2s[i], out_t)
    return jnp.transpose(out_t, (2, 0, 1))


# speedup vs baseline: 1.0077x; 1.0077x over previous
import functools

import jax
import jax.numpy as jnp
from jax import lax
from jax.experimental import pallas as pl
from jax.experimental.pallas import tpu as pltpu
from jax.experimental.pallas import tpu_sc as plsc

D = 64
BATCH = 4096
HIST = 200
HB = BATCH // 2
B = BATCH * HIST
SPLITS = (16, 56, 56, 56, 16)
assert sum(SPLITS) == HIST and all(s % 2 == 0 for s in SPLITS)
NSPLIT = len(SPLITS)
H0S = tuple(sum(SPLITS[:i]) for i in range(NSPLIT))
NC, NS = 2, 16
NW = NC * NS
CH = 256
assert HB % CH == 0


def _ring_depth(nch):
    for nb in (6, 5, 4, 3, 2):
        if nch % nb == 0:
            return nb
    return 1

_mesh = plsc.VectorSubcoreMesh(core_axis_name="c", subcore_axis_name="s")


def _make_sc_gather(h0, hn):
    bs = hn * BATCH
    b_per_w = bs // NW
    nch = b_per_w // CH
    assert b_per_w % CH == 0
    nb = _ring_depth(nch)

    @functools.partial(
        pl.kernel,
        mesh=_mesh,
        out_type=jax.ShapeDtypeStruct((bs // 2, 2 * D), jnp.float32),
        scratch_types=[
            pltpu.VMEM((b_per_w,), jnp.int32),
        ]
        + [pltpu.VMEM((CH, D), jnp.float32)] * nb
        + [pltpu.SemaphoreType.DMA] * (2 * nb),
        compiler_params=pltpu.CompilerParams(use_tc_tiling_on_sc=False),
    )
    def _sc_gather(idx_hbm, table_hbm, out_hbm, idx_all, *bufs):
        rows = bufs[:nb]
        sg = bufs[nb:2 * nb]
        sw = bufs[2 * nb:3 * nb]

        wid = lax.axis_index("s") * NC + lax.axis_index("c")
        base = h0 * BATCH + wid * b_per_w

        pltpu.sync_copy(idx_hbm.at[pl.ds(base, b_per_w)], idx_all)

        def gather_desc(c, b):
            return pltpu.make_async_copy(
                table_hbm.at[idx_all.at[pl.ds(c * CH, CH)]], rows[b], sg[b])

        def write_desc(c, b):
            q0 = base + c * CH
            h = q0 // BATCH - h0
            p0 = q0 % BATCH
            s = p0 // HB
            r0_ = p0 % HB
            drow = pl.multiple_of(h * HB + r0_, CH)
            dcol = pl.multiple_of(s * D, D)
            return pltpu.make_async_copy(
                rows[b], out_hbm.at[pl.ds(drow, CH), pl.ds(dcol, D)], sw[b])

        for b in range(nb):
            gather_desc(b, b).start()

        def body(it, carry):
            g = it * nb
            for b in range(nb):
                c = g + b
                gather_desc(c, b).wait()
                write_desc(c, b).start()
            for b in range(nb):
                c = g + b
                write_desc(c, b).wait()

                @pl.when(c + nb < nch)
                def _():
                    gather_desc(c + nb, b).start()

            return carry

        lax.fori_loop(0, nch // nb, body, 0)

    return _sc_gather


_sc_gather_halves = tuple(_make_sc_gather(H0S[i], SPLITS[i]) for i in range(NSPLIT))


def _tc_transpose_body_first(g_ref, o_ref):
    xt = g_ref[...].T
    o_ref[0, :, 0:HB] = xt[0:D, 0:HB]
    o_ref[0, :, HB:BATCH] = xt[D:2 * D, 0:HB]
    o_ref[1, :, 0:HB] = xt[0:D, HB:BATCH]
    o_ref[1, :, HB:BATCH] = xt[D:2 * D, HB:BATCH]


def _tc_transpose_body_rest(g_ref, _prev_ref, o_ref):
    _tc_transpose_body_first(g_ref, o_ref)


def _make_tc_transpose(split, first):
    kb = H0S[split] // 2
    grid = SPLITS[split] // 2
    if first:
        return pl.pallas_call(
            _tc_transpose_body_first,
            grid=(grid,),
            in_specs=[pl.BlockSpec((BATCH, 2 * D), lambda k: (k, 0))],
            out_specs=pl.BlockSpec((2, D, BATCH), lambda k: (k + kb, 0, 0)),
            out_shape=jax.ShapeDtypeStruct((HIST, D, BATCH), jnp.float32),
        )
    return pl.pallas_call(
        _tc_transpose_body_rest,
        grid=(grid,),
        in_specs=[
            pl.BlockSpec((BATCH, 2 * D), lambda k: (k, 0)),
            pl.BlockSpec(memory_space=pl.ANY),
        ],
        out_specs=pl.BlockSpec((2, D, BATCH), lambda k: (k + kb, 0, 0)),
        out_shape=jax.ShapeDtypeStruct((HIST, D, BATCH), jnp.float32),
        input_output_aliases={1: 0},
    )


_tc_first = _make_tc_transpose(0, True)
_tc_rest = tuple(_make_tc_transpose(i, False) for i in range(1, NSPLIT))


def kernel(matched_word_ids, word_embedding_weight):
    idx_t = matched_word_ids.T.reshape(-1).astype(jnp.int32)
    g2s = [f(idx_t, word_embedding_weight) for f in _sc_gather_halves]
    out_t = _tc_first(g2s[0])
    for i in range(1, NSPLIT):
        out_t = _tc_rest[i - 1](g2s[i], out_t)
    return jnp.transpose(out_t, (2, 0, 1))


# speedup vs baseline: 1.0167x; 1.0089x over previous
import functools

import jax
import jax.numpy as jnp
from jax import lax
from jax.experimental import pallas as pl
from jax.experimental.pallas import tpu as pltpu
from jax.experimental.pallas import tpu_sc as plsc

D = 64
BATCH = 4096
HIST = 200
HB = BATCH // 2
B = BATCH * HIST
SPLITS = (50, 50, 50, 50)
assert sum(SPLITS) == HIST and all(s % 2 == 0 for s in SPLITS)
NSPLIT = len(SPLITS)
H0S = tuple(sum(SPLITS[:i]) for i in range(NSPLIT))
NC, NS = 2, 16
NW = NC * NS
CH = 256
assert HB % CH == 0


def _ring_depth(nch):
    for nb in (6, 5, 4, 3, 2):
        if nch % nb == 0:
            return nb
    return 1

_mesh = plsc.VectorSubcoreMesh(core_axis_name="c", subcore_axis_name="s")


def _make_sc_gather(h0, hn):
    bs = hn * BATCH
    b_per_w = bs // NW
    nch = b_per_w // CH
    assert b_per_w % CH == 0
    nb = _ring_depth(nch)

    @functools.partial(
        pl.kernel,
        mesh=_mesh,
        out_type=jax.ShapeDtypeStruct((bs // 2, 2 * D), jnp.float32),
        scratch_types=[
            pltpu.VMEM((b_per_w,), jnp.int32),
        ]
        + [pltpu.VMEM((CH, D), jnp.float32)] * nb
        + [pltpu.SemaphoreType.DMA] * (2 * nb),
        compiler_params=pltpu.CompilerParams(use_tc_tiling_on_sc=False),
    )
    def _sc_gather(idx_hbm, table_hbm, out_hbm, idx_all, *bufs):
        rows = bufs[:nb]
        sg = bufs[nb:2 * nb]
        sw = bufs[2 * nb:3 * nb]

        wid = lax.axis_index("s") * NC + lax.axis_index("c")
        base = h0 * BATCH + wid * b_per_w

        pltpu.sync_copy(idx_hbm.at[pl.ds(base, b_per_w)], idx_all)

        def gather_desc(c, b):
            return pltpu.make_async_copy(
                table_hbm.at[idx_all.at[pl.ds(c * CH, CH)]], rows[b], sg[b])

        def write_desc(c, b):
            q0 = base + c * CH
            h = q0 // BATCH - h0
            p0 = q0 % BATCH
            s = p0 // HB
            r0_ = p0 % HB
            drow = pl.multiple_of(h * HB + r0_, CH)
            dcol = pl.multiple_of(s * D, D)
            return pltpu.make_async_copy(
                rows[b], out_hbm.at[pl.ds(drow, CH), pl.ds(dcol, D)], sw[b])

        for b in range(nb):
            gather_desc(b, b).start()

        def body(it, carry):
            g = it * nb
            for b in range(nb):
                c = g + b
                gather_desc(c, b).wait()
                write_desc(c, b).start()
            for b in range(nb):
                c = g + b
                write_desc(c, b).wait()

                @pl.when(c + nb < nch)
                def _():
                    gather_desc(c + nb, b).start()

            return carry

        lax.fori_loop(0, nch // nb, body, 0)

    return _sc_gather


_sc_gather_halves = tuple(_make_sc_gather(H0S[i], SPLITS[i]) for i in range(NSPLIT))


def _tc_transpose_body_first(g_ref, o_ref):
    xt = g_ref[...].T
    o_ref[0, :, 0:HB] = xt[0:D, 0:HB]
    o_ref[0, :, HB:BATCH] = xt[D:2 * D, 0:HB]
    o_ref[1, :, 0:HB] = xt[0:D, HB:BATCH]
    o_ref[1, :, HB:BATCH] = xt[D:2 * D, HB:BATCH]


def _tc_transpose_body_rest(g_ref, _prev_ref, o_ref):
    _tc_transpose_body_first(g_ref, o_ref)


def _make_tc_transpose(split, first):
    kb = H0S[split] // 2
    grid = SPLITS[split] // 2
    if first:
        return pl.pallas_call(
            _tc_transpose_body_first,
            grid=(grid,),
            in_specs=[pl.BlockSpec((BATCH, 2 * D), lambda k: (k, 0))],
            out_specs=pl.BlockSpec((2, D, BATCH), lambda k: (k + kb, 0, 0)),
            out_shape=jax.ShapeDtypeStruct((HIST, D, BATCH), jnp.float32),
        )
    return pl.pallas_call(
        _tc_transpose_body_rest,
        grid=(grid,),
        in_specs=[
            pl.BlockSpec((BATCH, 2 * D), lambda k: (k, 0)),
            pl.BlockSpec(memory_space=pl.ANY),
        ],
        out_specs=pl.BlockSpec((2, D, BATCH), lambda k: (k + kb, 0, 0)),
        out_shape=jax.ShapeDtypeStruct((HIST, D, BATCH), jnp.float32),
        input_output_aliases={1: 0},
    )


_tc_first = _make_tc_transpose(0, True)
_tc_rest = tuple(_make_tc_transpose(i, False) for i in range(1, NSPLIT))


def kernel(matched_word_ids, word_embedding_weight):
    idx_t = matched_word_ids.T.reshape(-1).astype(jnp.int32)
    g2s = [f(idx_t, word_embedding_weight) for f in _sc_gather_halves]
    out_t = _tc_first(g2s[0])
    for i in range(1, NSPLIT):
        out_t = _tc_rest[i - 1](g2s[i], out_t)
    return jnp.transpose(out_t, (2, 0, 1))
